# baseline (speedup 1.0000x reference)
import functools

import jax
import jax.numpy as jnp
from jax import lax
from jax.experimental import pallas as pl
from jax.experimental.pallas import tpu as pltpu
from jax.experimental.pallas import tpu_sc as plsc

_N, _K, _D, _OUT = 10000, 16, 256, 256
_L = 16
_NC, _NS = 2, 16
_NW = _NC * _NS
_CHUNK = 320
_NB = 8
_NCH = _D // _L
_DE = _D + _L


def _inv_body(feat_ref, inv_ref):
    x = feat_ref[...]
    inv_ref[...] = lax.rsqrt(jnp.sum(x * x, axis=1, keepdims=True))


_inv_call = pl.pallas_call(
    _inv_body,
    out_shape=jax.ShapeDtypeStruct((_N, 1), jnp.float32),
)


_mesh = plsc.VectorSubcoreMesh(core_axis_name="c", subcore_axis_name="s")


@functools.partial(
    pl.kernel,
    mesh=_mesh,
    compiler_params=pltpu.CompilerParams(needs_layout_passes=False),
    out_type=jax.ShapeDtypeStruct((_N, _DE), jnp.float32),
    scratch_types=[
        pltpu.VMEM((_N,), jnp.float32),
        pltpu.VMEM((_CHUNK * _K,), jnp.int32),
        pltpu.VMEM((_NB * _K, _D), jnp.float32),
        pltpu.VMEM((_NB * _K, _D), jnp.float32),
        pltpu.VMEM((_NB, _D), jnp.float32),
        pltpu.VMEM((_NB, _D), jnp.float32),
        pltpu.VMEM((_NB, _DE), jnp.float32),
        pltpu.VMEM((_NB, _DE), jnp.float32),
        pltpu.SemaphoreType.DMA,
        pltpu.SemaphoreType.DMA,
        pltpu.SemaphoreType.DMA,
        pltpu.SemaphoreType.DMA,
        pltpu.SemaphoreType.DMA,
        pltpu.SemaphoreType.DMA,
    ],
)
def _neigh_sc(feat_hbm, idxf_hbm, inv_hbm, agg_hbm,
              inv_v, idxall_v, rows_v0, rows_v1, cen_v0, cen_v1,
              acc_v0, acc_v1, gsem0, gsem1, csem0, csem1, osem0, osem1):
    rows_b = (rows_v0, rows_v1)
    cen_b = (cen_v0, cen_v1)
    acc_b = (acc_v0, acc_v1)
    gsem = (gsem0, gsem1)
    csem = (csem0, csem1)
    osem = (osem0, osem1)

    wid = lax.axis_index("s") * _NC + lax.axis_index("c")
    base = wid * _CHUNK
    rows_w = jnp.minimum(_CHUNK, _N - base)
    nblk = rows_w // _NB
    start = jnp.minimum(base, _N - _CHUNK)
    off = base - start

    pltpu.sync_copy(inv_hbm, inv_v)
    pltpu.sync_copy(idxf_hbm.at[pl.ds(start * _K, _CHUNK * _K)], idxall_v)

    def start_cen(blk, b):
        rowbase = base + blk * _NB
        pltpu.async_copy(feat_hbm.at[pl.ds(rowbase, _NB)], cen_b[b], csem[b])

    def start_gather(blk, b):
        lo = (off + blk * _NB) * _K
        pltpu.async_copy(feat_hbm.at[idxall_v.at[pl.ds(lo, _NB * _K)]],
                         rows_b[b], gsem[b])

    start_cen(0, 0)
    start_gather(0, 0)

    @pl.loop(0, nblk, step=2)
    def _pair(g):
        for b in (0, 1):
            blk = g + b
            rowbase = base + blk * _NB
            nxt = 1 - b

            @pl.when(blk + 1 < nblk)
            def _prefetch():
                start_cen(blk + 1, nxt)
                start_gather(blk + 1, nxt)

            pltpu.make_async_copy(feat_hbm.at[pl.ds(0, _NB * _K)],
                                  rows_b[b], gsem[b]).wait()
            pltpu.make_async_copy(feat_hbm.at[pl.ds(0, _NB)],
                                  cen_b[b], csem[b]).wait()

            @pl.when(blk >= 2)
            def _drain_out():
                pltpu.make_async_copy(acc_b[b], agg_hbm.at[pl.ds(rowbase, _NB)],
                                      osem[b]).wait()

            def node_body(j, c2):
                r0 = j * _K
                kidx = idxall_v[pl.ds((off + blk * _NB + j) * _K, _K)]
                invk = plsc.load_gather(inv_v, [kidx])
                ctr_idx = jnp.full((_L,), rowbase + j, dtype=jnp.int32)
                inv_n = plsc.load_gather(inv_v, [ctr_idx])
                wscale = invk * inv_n
                cen = [cen_b[b][j, pl.ds(i * _L, _L)] for i in range(_NCH)]
                for k in range(_K):
                    row = [rows_b[b][r0 + k, pl.ds(i * _L, _L)]
                           for i in range(_NCH)]
                    p = row[0] * cen[0]
                    for i in range(1, _NCH):
                        p = p + row[i] * cen[i]
                    w = jnp.sum(p) * wscale[k]
                    wb = jnp.full((_L,), w)
                    if k == 0:
                        for i in range(_NCH):
                            acc_b[b][j, pl.ds(i * _L, _L)] = w * row[i]
                        acc_b[b][j, pl.ds(_D, _L)] = wb
                    else:
                        for i in range(_NCH):
                            plsc.addupdate(
                                acc_b[b].at[j, pl.ds(i * _L, _L)], w * row[i])
                        plsc.addupdate(acc_b[b].at[j, pl.ds(_D, _L)], wb)
                return c2

            lax.fori_loop(0, _NB, node_body, 0)
            pltpu.async_copy(acc_b[b], agg_hbm.at[pl.ds(rowbase, _NB)], osem[b])

    rb0 = base + (nblk - 2) * _NB
    rb1 = base + (nblk - 1) * _NB
    pltpu.make_async_copy(acc_v0, agg_hbm.at[pl.ds(rb0, _NB)], osem0).wait()
    pltpu.make_async_copy(acc_v1, agg_hbm.at[pl.ds(rb1, _NB)], osem1).wait()


def _fin_body(agg_ref, feat_ref, w_ref, b_ref, out_ref):
    w1 = w_ref[:, :_D]
    w2 = w_ref[:, _D:]
    agg = agg_ref[:, :_D]
    sw = agg_ref[:, _D:_D + 1]
    dn = (((1,), (1,)), ((), ()))
    p = lax.dot_general(feat_ref[...], w2, dn,
                        preferred_element_type=jnp.float32) + b_ref[...]
    a = lax.dot_general(agg, w1, dn,
                        preferred_element_type=jnp.float32)
    out_ref[...] = (a + sw * p) * (1.0 / _K)


_fin_call = pl.pallas_call(
    _fin_body,
    out_shape=jax.ShapeDtypeStruct((_N, _OUT), jnp.float32),
)


def kernel(feat_prop, neigh_idx, W, b):
    idx_flat = neigh_idx.astype(jnp.int32).reshape(-1)
    inv = _inv_call(feat_prop)
    agg_ext = _neigh_sc(feat_prop, idx_flat, inv.reshape(_N))
    return _fin_call(agg_ext, feat_prop, W, b.reshape(1, _OUT))
